# baseline (speedup 1.0000x reference)
import functools

import jax
import jax.numpy as jnp
from jax import lax
from jax.experimental import pallas as pl
from jax.experimental.pallas import tpu as pltpu
from jax.experimental.pallas import tpu_sc as plsc

N = 10000
E = 320000
D = 128
CD = D // 2
NC = 2
NS = 16
EB = 80
BPT = E // (NS * EB)
NCH = 10
IB = BPT // NCH
NBUF = 12
RPT_MAIN = 640
RPT_LAST = N - 15 * RPT_MAIN

_mesh = plsc.VectorSubcoreMesh(core_axis_name="c", subcore_axis_name="s")


@functools.partial(
    pl.kernel,
    out_type=jax.ShapeDtypeStruct((NC, 1, 10240), jnp.float32),
    mesh=_mesh,
    scratch_types=[
        pltpu.VMEM((IB, EB), jnp.int32),
        pltpu.VMEM((EB,), jnp.float32),
        pltpu.VMEM((640,), jnp.float32),
        pltpu.VMEM_SHARED((10240,), jnp.float32),
    ],
)
def _deg_kernel(dst_hbm, out_hbm, dst_v, ones_v, zeros_v, hist_sh):
    c = lax.axis_index("c")
    s = lax.axis_index("s")
    r0 = s * 640

    one16 = jnp.full((16,), 1.0, jnp.float32)
    zero16 = jnp.zeros((16,), jnp.float32)
    for i in list(range(0, EB - 16, 16)) + [EB - 16]:
        ones_v[pl.ds(i, 16)] = one16
    for i in range(640 // 16):
        zeros_v[pl.ds(i * 16, 16)] = zero16

    pltpu.sync_copy(zeros_v, hist_sh.at[pl.ds(r0, 640)])
    plsc.subcore_barrier()

    def chunk(i, carry):
        pltpu.sync_copy(dst_hbm.at[s, i * 2 + c], dst_v)

        def body(j, _):
            pltpu.sync_copy(ones_v, hist_sh.at[dst_v.at[j]], add=True)
            return _

        lax.fori_loop(0, IB, body, 0)
        return carry

    lax.fori_loop(0, NCH // 2, chunk, 0)
    plsc.subcore_barrier()

    pltpu.sync_copy(hist_sh.at[pl.ds(r0, 640)],
                    out_hbm.at[c, 0, pl.ds(r0, 640)])


@functools.partial(
    pl.kernel,
    out_type=jax.ShapeDtypeStruct((NC, N, CD), jnp.float32),
    mesh=_mesh,
    scratch_types=[
        pltpu.VMEM((IB, EB), jnp.int32),
        pltpu.VMEM((IB, EB), jnp.int32),
    ] + [pltpu.VMEM((EB, CD), jnp.float32) for _ in range(NBUF)] + [
        pltpu.VMEM_SHARED((N, CD), jnp.float32),
    ] + [pltpu.SemaphoreType.DMA for _ in range(NBUF)],
    compiler_params=pltpu.CompilerParams(use_tc_tiling_on_sc=False),
)
def _agg_kernel(gl_hbm, gr_hbm, src_hbm, dst_hbm, out_hbm, src_v, dst_v,
                *rest):
    bufs = rest[:NBUF]
    acc_sh = rest[NBUF]
    sems = rest[NBUF + 1:]
    c = lax.axis_index("c")
    s = lax.axis_index("s")
    r0 = s * RPT_MAIN

    zero16 = jnp.zeros((16,), jnp.float32)

    def zrow(i, carry):
        for j in range(CD // 16):
            bufs[0][i, pl.ds(j * 16, 16)] = zero16
        return carry

    lax.fori_loop(0, EB, zrow, 0)

    @pl.when(s < 15)
    def _():
        for j in range(RPT_MAIN // EB):
            pltpu.sync_copy(bufs[0], acc_sh.at[pl.ds(r0 + j * EB, EB)])

    @pl.when(s == 15)
    def _():
        for j in range(RPT_LAST // EB):
            pltpu.sync_copy(bufs[0],
                            acc_sh.at[pl.ds(15 * RPT_MAIN + j * EB, EB)])

    plsc.subcore_barrier()

    def run(g_ref):
        def chunk(ci, carry):
            pltpu.sync_copy(src_hbm.at[s, ci], src_v)
            pltpu.sync_copy(dst_hbm.at[s, ci], dst_v)
            for k in range(NBUF - 1):
                pltpu.async_copy(g_ref.at[src_v.at[k]], bufs[k], sems[k])

            def step(j, t):
                @pl.when(j < IB)
                def _():
                    pltpu.make_async_copy(g_ref.at[src_v.at[j]], bufs[t],
                                          sems[t]).wait()

                    @pl.when(j + NBUF - 1 < IB)
                    def _():
                        tn = (t - 1) % NBUF
                        pltpu.async_copy(g_ref.at[src_v.at[j + NBUF - 1]],
                                         bufs[tn], sems[tn])

                    pltpu.sync_copy(bufs[t], acc_sh.at[dst_v.at[j]],
                                    add=True)

            def body(i, _):
                j0 = i * NBUF
                for t in range(NBUF):
                    step(j0 + t, t)
                return _

            lax.fori_loop(0, (IB + NBUF - 1) // NBUF, body, 0)
            return carry

        lax.fori_loop(0, NCH, chunk, 0)

    @pl.when(c == 0)
    def _():
        run(gl_hbm)

    @pl.when(c == 1)
    def _():
        run(gr_hbm)

    plsc.subcore_barrier()

    @pl.when(s < 15)
    def _():
        pltpu.sync_copy(acc_sh.at[pl.ds(r0, RPT_MAIN)],
                        out_hbm.at[c, pl.ds(r0, RPT_MAIN)])

    @pl.when(s == 15)
    def _():
        pltpu.sync_copy(acc_sh.at[pl.ds(15 * RPT_MAIN, RPT_LAST)],
                        out_hbm.at[c, pl.ds(15 * RPT_MAIN, RPT_LAST)])


RB = 10
BLK = N // RB


def _row_spec(shape_cols):
    return pl.BlockSpec((BLK, shape_cols), lambda i: (i, 0))


def _plane_spec(p):
    return pl.BlockSpec((1, BLK, CD), lambda i, _p=p: (_p, i, 0))


_W_SPEC = pl.BlockSpec((D, D), lambda i: (0, 0))
_B_SPEC = pl.BlockSpec((1, D), lambda i: (0, 0))


def _mm0_body(x_ref, w_ref, d0_ref, d1_ref, gl_ref, gr_ref, dinv_ref):
    dinv = lax.rsqrt(d0_ref[...] + d1_ref[...] + 1.0)
    dinv_ref[...] = dinv
    g = jnp.dot(x_ref[...], w_ref[...],
                preferred_element_type=jnp.float32) * dinv
    gl_ref[...] = g[:, :CD]
    gr_ref[...] = g[:, CD:]


def _assemble(pl_ref, pr_ref, gl_ref, gr_ref):
    return jnp.concatenate(
        [pl_ref[0] + gl_ref[...], pr_ref[0] + gr_ref[...]], axis=1)


def _post_body(pl_ref, pr_ref, gl_ref, gr_ref, dinv_ref, b_ref, w_ref,
               y_ref, gnl_ref, gnr_ref):
    dinv = dinv_ref[...]
    y = jnp.maximum(dinv * _assemble(pl_ref, pr_ref, gl_ref, gr_ref)
                    + b_ref[...], 0.0)
    y_ref[...] = y
    gn = jnp.dot(y, w_ref[...], preferred_element_type=jnp.float32) * dinv
    gnl_ref[...] = gn[:, :CD]
    gnr_ref[...] = gn[:, CD:]


def _post_res_body(pl_ref, pr_ref, gl_ref, gr_ref, dinv_ref, b_ref, res_ref,
                   w_ref, gnl_ref, gnr_ref):
    dinv = dinv_ref[...]
    y = jnp.maximum(dinv * _assemble(pl_ref, pr_ref, gl_ref, gr_ref)
                    + b_ref[...] + res_ref[...], 0.0)
    gn = jnp.dot(y, w_ref[...], preferred_element_type=jnp.float32) * dinv
    gnl_ref[...] = gn[:, :CD]
    gnr_ref[...] = gn[:, CD:]


def _head_body(pl_ref, pr_ref, gl_ref, gr_ref, dinv_ref, b_ref, res_ref,
               w1_ref, b1_ref, w2_ref, b2_ref, w3_ref, b3_ref, o_ref):
    y = jnp.maximum(dinv_ref[...] * _assemble(pl_ref, pr_ref, gl_ref, gr_ref)
                    + b_ref[...] + res_ref[...], 0.0)
    h1 = jnp.maximum(jnp.dot(y, w1_ref[...],
                             preferred_element_type=jnp.float32)
                     + b1_ref[...], 0.0)
    h2 = jnp.maximum(jnp.dot(h1, w2_ref[...],
                             preferred_element_type=jnp.float32)
                     + b2_ref[...], 0.0)
    z = jnp.dot(h2, w3_ref[...], preferred_element_type=jnp.float32) + b3_ref[...]
    mask = lax.broadcasted_iota(jnp.int32, (BLK, D), 1) < 7
    zm = jnp.where(mask, z, -jnp.inf)
    mx = jnp.max(zm, axis=1, keepdims=True)
    ez = jnp.where(mask, jnp.exp(z - mx), 0.0)
    lse = jnp.log(jnp.sum(ez, axis=1, keepdims=True)) + mx
    o_ref[...] = z - lse


def kernel(x, edge_index, Wc0, bc0, Wc1, bc1, Wc2, bc2, Wf1, bf1, Wf2, bf2,
           Wf3, bf3):
    f32 = jnp.float32
    src = edge_index[0].reshape(NS, NCH, IB, EB)
    dst = edge_index[1].reshape(NS, NCH, IB, EB)

    deg_parts = _deg_kernel(dst)
    d0 = deg_parts[0, 0, :N].reshape(N, 1)
    d1 = deg_parts[1, 0, :N].reshape(N, 1)

    gl0, gr0, dinv = pl.pallas_call(
        _mm0_body,
        grid=(RB,),
        in_specs=[_row_spec(D), _W_SPEC, _row_spec(1), _row_spec(1)],
        out_specs=[_row_spec(CD), _row_spec(CD), _row_spec(1)],
        out_shape=[jax.ShapeDtypeStruct((N, CD), f32),
                   jax.ShapeDtypeStruct((N, CD), f32),
                   jax.ShapeDtypeStruct((N, 1), f32)],
    )(x, Wc0, d0, d1)

    p = _agg_kernel(gl0, gr0, src, dst)
    y1, gl1, gr1 = pl.pallas_call(
        _post_body,
        grid=(RB,),
        in_specs=[_plane_spec(0), _plane_spec(1), _row_spec(CD),
                  _row_spec(CD), _row_spec(1), _B_SPEC, _W_SPEC],
        out_specs=[_row_spec(D), _row_spec(CD), _row_spec(CD)],
        out_shape=[jax.ShapeDtypeStruct((N, D), f32),
                   jax.ShapeDtypeStruct((N, CD), f32),
                   jax.ShapeDtypeStruct((N, CD), f32)],
    )(p, p, gl0, gr0, dinv, bc0.reshape(1, D), Wc1)

    p = _agg_kernel(gl1, gr1, src, dst)
    gl2, gr2 = pl.pallas_call(
        _post_res_body,
        grid=(RB,),
        in_specs=[_plane_spec(0), _plane_spec(1), _row_spec(CD),
                  _row_spec(CD), _row_spec(1), _B_SPEC, _row_spec(D),
                  _W_SPEC],
        out_specs=[_row_spec(CD), _row_spec(CD)],
        out_shape=[jax.ShapeDtypeStruct((N, CD), f32),
                   jax.ShapeDtypeStruct((N, CD), f32)],
    )(p, p, gl1, gr1, dinv, bc1.reshape(1, D), y1, Wc2)

    p = _agg_kernel(gl2, gr2, src, dst)
    w3p = jnp.pad(Wf3, ((0, 0), (0, D - 7)))
    b3p = jnp.pad(bf3, (0, D - 7)).reshape(1, D)
    out = pl.pallas_call(
        _head_body,
        grid=(RB,),
        in_specs=[_plane_spec(0), _plane_spec(1), _row_spec(CD),
                  _row_spec(CD), _row_spec(1), _B_SPEC, _row_spec(D),
                  _W_SPEC, _B_SPEC, _W_SPEC, _B_SPEC, _W_SPEC, _B_SPEC],
        out_specs=_row_spec(D),
        out_shape=jax.ShapeDtypeStruct((N, D), f32),
    )(p, p, gl2, gr2, dinv, bc2.reshape(1, D), y1,
      Wf1, bf1.reshape(1, D), Wf2, bf2.reshape(1, D), w3p, b3p)
    return out[:, :7]


# speedup vs baseline: 1.3121x; 1.3121x over previous
import functools

import jax
import jax.numpy as jnp
from jax import lax
from jax.experimental import pallas as pl
from jax.experimental.pallas import tpu as pltpu
from jax.experimental.pallas import tpu_sc as plsc

N = 10000
E = 320000
D = 128
NC = 2
NS = 16
NW = NC * NS
EB = 80
BPT = E // (NW * EB)
NCH = 5
IB = BPT // NCH
NBUF = 8
RPT_MAIN = 640
RPT_LAST = N - 15 * RPT_MAIN
RB = 10
BLK = N // RB

_mesh = plsc.VectorSubcoreMesh(core_axis_name="c", subcore_axis_name="s")


@functools.partial(
    pl.kernel,
    out_type=jax.ShapeDtypeStruct((NC, 1, 10240), jnp.float32),
    mesh=_mesh,
    scratch_types=[
        pltpu.VMEM((IB, EB), jnp.int32),
        pltpu.VMEM((EB,), jnp.float32),
        pltpu.VMEM((640,), jnp.float32),
        pltpu.VMEM_SHARED((10240,), jnp.float32),
    ],
)
def _deg_kernel(dst_hbm, out_hbm, dst_v, ones_v, zeros_v, hist_sh):
    c = lax.axis_index("c")
    s = lax.axis_index("s")
    w = s * NC + c
    r0 = s * 640

    one16 = jnp.full((16,), 1.0, jnp.float32)
    zero16 = jnp.zeros((16,), jnp.float32)
    for i in list(range(0, EB - 16, 16)) + [EB - 16]:
        ones_v[pl.ds(i, 16)] = one16
    for i in range(640 // 16):
        zeros_v[pl.ds(i * 16, 16)] = zero16

    pltpu.sync_copy(zeros_v, hist_sh.at[pl.ds(r0, 640)])
    plsc.subcore_barrier()

    def chunk(ci, carry):
        pltpu.sync_copy(dst_hbm.at[w, ci], dst_v)

        def body(j, _):
            pltpu.sync_copy(ones_v, hist_sh.at[dst_v.at[j]], add=True)
            return _

        lax.fori_loop(0, IB, body, 0)
        return carry

    lax.fori_loop(0, NCH, chunk, 0)
    plsc.subcore_barrier()

    pltpu.sync_copy(hist_sh.at[pl.ds(r0, 640)],
                    out_hbm.at[c, 0, pl.ds(r0, 640)])


@functools.partial(
    pl.kernel,
    out_type=jax.ShapeDtypeStruct((NC, N, D), jnp.bfloat16),
    mesh=_mesh,
    scratch_types=[
        pltpu.VMEM((IB, EB), jnp.int32),
        pltpu.VMEM((IB, EB), jnp.int32),
    ] + [pltpu.VMEM((EB, D), jnp.bfloat16) for _ in range(NBUF)] + [
        pltpu.VMEM_SHARED((N, D), jnp.bfloat16),
    ] + [pltpu.SemaphoreType.DMA for _ in range(NBUF)],
    compiler_params=pltpu.CompilerParams(use_tc_tiling_on_sc=False),
)
def _agg_kernel(g_hbm, src_hbm, dst_hbm, out_hbm, src_v, dst_v, *rest):
    bufs = rest[:NBUF]
    acc_sh = rest[NBUF]
    sems = rest[NBUF + 1:]
    c = lax.axis_index("c")
    s = lax.axis_index("s")
    w = s * NC + c
    r0 = s * RPT_MAIN

    zero32 = jnp.zeros((32,), jnp.bfloat16)

    def zrow(i, carry):
        for j in range(D // 32):
            bufs[0][i, pl.ds(j * 32, 32)] = zero32
        return carry

    lax.fori_loop(0, EB, zrow, 0)

    @pl.when(s < 15)
    def _():
        for j in range(RPT_MAIN // EB):
            pltpu.sync_copy(bufs[0], acc_sh.at[pl.ds(r0 + j * EB, EB)])

    @pl.when(s == 15)
    def _():
        for j in range(RPT_LAST // EB):
            pltpu.sync_copy(bufs[0],
                            acc_sh.at[pl.ds(15 * RPT_MAIN + j * EB, EB)])

    plsc.subcore_barrier()

    def chunk(ci, carry):
        pltpu.sync_copy(src_hbm.at[w, ci], src_v)
        pltpu.sync_copy(dst_hbm.at[w, ci], dst_v)
        for k in range(NBUF - 1):
            pltpu.async_copy(g_hbm.at[src_v.at[k]], bufs[k], sems[k])

        def step(j, t):
            @pl.when(j < IB)
            def _():
                pltpu.make_async_copy(g_hbm.at[src_v.at[j]], bufs[t],
                                      sems[t]).wait()

                @pl.when(j + NBUF - 1 < IB)
                def _():
                    tn = (t - 1) % NBUF
                    pltpu.async_copy(g_hbm.at[src_v.at[j + NBUF - 1]],
                                     bufs[tn], sems[tn])

                pltpu.sync_copy(bufs[t], acc_sh.at[dst_v.at[j]], add=True)

        def body(i, _):
            j0 = i * NBUF
            for t in range(NBUF):
                step(j0 + t, t)
            return _

        lax.fori_loop(0, (IB + NBUF - 1) // NBUF, body, 0)
        return carry

    lax.fori_loop(0, NCH, chunk, 0)
    plsc.subcore_barrier()

    @pl.when(s < 15)
    def _():
        pltpu.sync_copy(acc_sh.at[pl.ds(r0, RPT_MAIN)],
                        out_hbm.at[c, pl.ds(r0, RPT_MAIN)])

    @pl.when(s == 15)
    def _():
        pltpu.sync_copy(acc_sh.at[pl.ds(15 * RPT_MAIN, RPT_LAST)],
                        out_hbm.at[c, pl.ds(15 * RPT_MAIN, RPT_LAST)])


def _row_spec(shape_cols):
    return pl.BlockSpec((BLK, shape_cols), lambda i: (i, 0))


def _plane_spec(p):
    return pl.BlockSpec((1, BLK, D), lambda i, _p=p: (_p, i, 0))

_W_SPEC = pl.BlockSpec((D, D), lambda i: (0, 0))
_B_SPEC = pl.BlockSpec((1, D), lambda i: (0, 0))


def _mm0_body(x_ref, w_ref, d0_ref, d1_ref, g_ref, gb_ref, dinv_ref):
    dinv = lax.rsqrt(d0_ref[...] + d1_ref[...] + 1.0)
    dinv_ref[...] = dinv
    g = jnp.dot(x_ref[...], w_ref[...],
                preferred_element_type=jnp.float32) * dinv
    g_ref[...] = g
    gb_ref[...] = g.astype(jnp.bfloat16)


def _psum(p0_ref, p1_ref, g_ref):
    return (p0_ref[0].astype(jnp.float32) + p1_ref[0].astype(jnp.float32)
            + g_ref[...])


def _post_body(p0_ref, p1_ref, g_ref, dinv_ref, b_ref, w_ref, y_ref, gn_ref,
               gnb_ref):
    dinv = dinv_ref[...]
    y = jnp.maximum(dinv * _psum(p0_ref, p1_ref, g_ref) + b_ref[...], 0.0)
    y_ref[...] = y
    gn = jnp.dot(y, w_ref[...], preferred_element_type=jnp.float32) * dinv
    gn_ref[...] = gn
    gnb_ref[...] = gn.astype(jnp.bfloat16)


def _post_res_body(p0_ref, p1_ref, g_ref, dinv_ref, b_ref, res_ref, w_ref,
                   gn_ref, gnb_ref):
    dinv = dinv_ref[...]
    y = jnp.maximum(dinv * _psum(p0_ref, p1_ref, g_ref)
                    + b_ref[...] + res_ref[...], 0.0)
    gn = jnp.dot(y, w_ref[...], preferred_element_type=jnp.float32) * dinv
    gn_ref[...] = gn
    gnb_ref[...] = gn.astype(jnp.bfloat16)


def _head_body(p0_ref, p1_ref, g_ref, dinv_ref, b_ref, res_ref,
               w1_ref, b1_ref, w2_ref, b2_ref, w3_ref, b3_ref, o_ref):
    y = jnp.maximum(dinv_ref[...] * _psum(p0_ref, p1_ref, g_ref)
                    + b_ref[...] + res_ref[...], 0.0)
    h1 = jnp.maximum(jnp.dot(y, w1_ref[...],
                             preferred_element_type=jnp.float32)
                     + b1_ref[...], 0.0)
    h2 = jnp.maximum(jnp.dot(h1, w2_ref[...],
                             preferred_element_type=jnp.float32)
                     + b2_ref[...], 0.0)
    z = jnp.dot(h2, w3_ref[...], preferred_element_type=jnp.float32) + b3_ref[...]
    mask = lax.broadcasted_iota(jnp.int32, (BLK, D), 1) < 7
    zm = jnp.where(mask, z, -jnp.inf)
    mx = jnp.max(zm, axis=1, keepdims=True)
    ez = jnp.where(mask, jnp.exp(z - mx), 0.0)
    lse = jnp.log(jnp.sum(ez, axis=1, keepdims=True)) + mx
    o_ref[...] = z - lse


def kernel(x, edge_index, Wc0, bc0, Wc1, bc1, Wc2, bc2, Wf1, bf1, Wf2, bf2,
           Wf3, bf3):
    f32 = jnp.float32
    src = edge_index[0].reshape(NW, NCH, IB, EB)
    dst = edge_index[1].reshape(NW, NCH, IB, EB)

    deg_parts = _deg_kernel(dst)
    d0 = deg_parts[0, 0, :N].reshape(N, 1)
    d1 = deg_parts[1, 0, :N].reshape(N, 1)

    g0, gb0, dinv = pl.pallas_call(
        _mm0_body,
        grid=(RB,),
        in_specs=[_row_spec(D), _W_SPEC, _row_spec(1), _row_spec(1)],
        out_specs=[_row_spec(D), _row_spec(D), _row_spec(1)],
        out_shape=[jax.ShapeDtypeStruct((N, D), f32),
                   jax.ShapeDtypeStruct((N, D), jnp.bfloat16),
                   jax.ShapeDtypeStruct((N, 1), f32)],
    )(x, Wc0, d0, d1)

    p = _agg_kernel(gb0, src, dst)
    y1, g1, gb1 = pl.pallas_call(
        _post_body,
        grid=(RB,),
        in_specs=[_plane_spec(0), _plane_spec(1), _row_spec(D), _row_spec(1),
                  _B_SPEC, _W_SPEC],
        out_specs=[_row_spec(D), _row_spec(D), _row_spec(D)],
        out_shape=[jax.ShapeDtypeStruct((N, D), f32),
                   jax.ShapeDtypeStruct((N, D), f32),
                   jax.ShapeDtypeStruct((N, D), jnp.bfloat16)],
    )(p, p, g0, dinv, bc0.reshape(1, D), Wc1)

    p = _agg_kernel(gb1, src, dst)
    g2, gb2 = pl.pallas_call(
        _post_res_body,
        grid=(RB,),
        in_specs=[_plane_spec(0), _plane_spec(1), _row_spec(D), _row_spec(1),
                  _B_SPEC, _row_spec(D), _W_SPEC],
        out_specs=[_row_spec(D), _row_spec(D)],
        out_shape=[jax.ShapeDtypeStruct((N, D), f32),
                   jax.ShapeDtypeStruct((N, D), jnp.bfloat16)],
    )(p, p, g1, dinv, bc1.reshape(1, D), y1, Wc2)

    p = _agg_kernel(gb2, src, dst)
    w3p = jnp.pad(Wf3, ((0, 0), (0, D - 7)))
    b3p = jnp.pad(bf3, (0, D - 7)).reshape(1, D)
    out = pl.pallas_call(
        _head_body,
        grid=(RB,),
        in_specs=[_plane_spec(0), _plane_spec(1), _row_spec(D), _row_spec(1),
                  _B_SPEC, _row_spec(D),
                  _W_SPEC, _B_SPEC, _W_SPEC, _B_SPEC, _W_SPEC, _B_SPEC],
        out_specs=_row_spec(D),
        out_shape=jax.ShapeDtypeStruct((N, D), f32),
    )(p, p, g2, dinv, bc2.reshape(1, D), y1,
      Wf1, bf1.reshape(1, D), Wf2, bf2.reshape(1, D), w3p, b3p)
    return out[:, :7]
